# baseline (speedup 1.0000x reference)
import functools

import jax
import jax.numpy as jnp
from jax import lax
from jax.experimental import pallas as pl
from jax.experimental.pallas import tpu as pltpu
from jax.experimental.pallas import tpu_sc as plsc

K = 5
KK = K * K
N = 10000
E = 160000
NG = 64
N_PAD = 10240
E_PAD = 163840
NW = 32
EW = E_PAD // NW
B = 128
NB = EW // B
ROWS = E_PAD // B
VROWS = E // B
NSL = N_PAD // 16


def _prep(p0, p1, src):
    def body(p0_r, p1_r, src_r, e_r):
        i = pl.program_id(0)
        rows = i * 128 + lax.broadcasted_iota(jnp.int32, (128, 128), 0)
        val = (rows < VROWS).astype(jnp.float32)
        a = p0_r[...] * (K - 1.0)
        ia = jnp.clip(jnp.floor(a), 0.0, K - 2.0)
        fa = a - ia
        b = p1_r[...] * (K - 1.0)
        ib = jnp.clip(jnp.floor(b), 0.0, K - 2.0)
        fb = b - ib
        bc = lambda v: lax.bitcast_convert_type(v, jnp.int32)
        base = src_r[...] * KK + ia.astype(jnp.int32) * K + ib.astype(jnp.int32)
        e_r[:, 0, :] = base
        e_r[:, 1, :] = base + 1
        e_r[:, 2, :] = base + K
        e_r[:, 3, :] = base + K + 1
        e_r[:, 4, :] = bc((1.0 - fa) * (1.0 - fb) * val)
        e_r[:, 5, :] = bc((1.0 - fa) * fb * val)
        e_r[:, 6, :] = bc(fa * (1.0 - fb) * val)
        e_r[:, 7, :] = bc(fa * fb * val)
        e_r[:, 8, :] = bc(val)

    spec = pl.BlockSpec((128, 128), lambda i: (i, 0))
    ospec = pl.BlockSpec((128, 9, 128), lambda i: (i, 0, 0))
    return pl.pallas_call(
        body, grid=(ROWS // 128,),
        in_specs=[spec] * 3, out_specs=ospec,
        out_shape=jax.ShapeDtypeStruct((ROWS, 9, 128), jnp.int32),
    )(p0, p1, src)


def _mm(h, wflat):
    n, cin = h.shape
    cols = wflat.shape[1]
    blk = 512

    def body(h_r, w_r, o_r):
        o_r[...] = jnp.dot(h_r[...], w_r[...], preferred_element_type=jnp.float32)

    return pl.pallas_call(
        body, grid=(n // blk,),
        in_specs=[pl.BlockSpec((blk, cin), lambda i: (i, 0)),
                  pl.BlockSpec((cin, cols), lambda i: (0, 0))],
        out_specs=pl.BlockSpec((blk, cols), lambda i: (i, 0)),
        out_shape=jax.ShapeDtypeStruct((n, cols), jnp.float32),
    )(h, wflat)


def _post1(aggp, h_prev, root, bias):
    cout = root.shape[1]
    M = aggp.shape[2]
    cin = h_prev.shape[1]
    blk = 1024

    def body(a_r, h_r, r_r, b_r, o_r, d_r):
        a = a_r[0] + a_r[1]
        feat = a[:, :cout]
        cnt = a[:, cout:cout + 1]
        v = (feat / jnp.maximum(cnt, 1.0)
             + jnp.dot(h_r[...], r_r[...], preferred_element_type=jnp.float32)
             + b_r[...])
        o_r[...] = jnp.where(v > 0.0, v, jnp.exp(jnp.minimum(v, 0.0)) - 1.0)
        d_r[...] = cnt

    return pl.pallas_call(
        body, grid=(N_PAD // blk,),
        in_specs=[
            pl.BlockSpec((2, blk, M), lambda i: (0, i, 0)),
            pl.BlockSpec((blk, cin), lambda i: (i, 0)),
            pl.BlockSpec((cin, cout), lambda i: (0, 0)),
            pl.BlockSpec((1, cout), lambda i: (0, 0)),
        ],
        out_specs=[pl.BlockSpec((blk, cout), lambda i: (i, 0)),
                   pl.BlockSpec((blk, 1), lambda i: (i, 0))],
        out_shape=[jax.ShapeDtypeStruct((N_PAD, cout), jnp.float32),
                   jax.ShapeDtypeStruct((N_PAD, 1), jnp.float32)],
    )(aggp, h_prev, root, bias)


def _post(aggp, deg, h_prev, root, bias):
    cout = root.shape[1]
    M = aggp.shape[2]
    cin = h_prev.shape[1]
    blk = 1024

    def body(a_r, d_r, h_r, r_r, b_r, o_r):
        feat = a_r[0] + a_r[1]
        v = (feat / jnp.maximum(d_r[...], 1.0)
             + jnp.dot(h_r[...], r_r[...], preferred_element_type=jnp.float32)
             + b_r[...])
        o_r[...] = jnp.where(v > 0.0, v, jnp.exp(jnp.minimum(v, 0.0)) - 1.0)

    return pl.pallas_call(
        body, grid=(N_PAD // blk,),
        in_specs=[
            pl.BlockSpec((2, blk, M), lambda i: (0, i, 0)),
            pl.BlockSpec((blk, 1), lambda i: (i, 0)),
            pl.BlockSpec((blk, cin), lambda i: (i, 0)),
            pl.BlockSpec((cin, cout), lambda i: (0, 0)),
            pl.BlockSpec((1, cout), lambda i: (0, 0)),
        ],
        out_specs=pl.BlockSpec((blk, cout), lambda i: (i, 0)),
        out_shape=jax.ShapeDtypeStruct((N_PAD, cout), jnp.float32),
    )(aggp, deg, h_prev, root, bias)


def _pool(h3, bat, fc_w, fc_b):
    def body(h_r, b_r, w_r, fb_r, o_r):
        gi = lax.broadcasted_iota(jnp.int32, (NG, N_PAD), 0)
        oh = (gi == b_r[...]).astype(jnp.float32)
        seg = jnp.dot(oh, h_r[...], preferred_element_type=jnp.float32)
        cnt = jnp.sum(oh, axis=1, keepdims=True)
        g = seg / jnp.maximum(cnt, 1.0)
        logits = jnp.dot(g, w_r[...], preferred_element_type=jnp.float32) + fb_r[...]
        m = jnp.max(logits, axis=1, keepdims=True)
        lse = jnp.log(jnp.sum(jnp.exp(logits - m), axis=1, keepdims=True)) + m
        o_r[...] = logits - lse

    return pl.pallas_call(
        body,
        in_specs=[pl.BlockSpec((N_PAD, 64), lambda: (0, 0)),
                  pl.BlockSpec((1, N_PAD), lambda: (0, 0)),
                  pl.BlockSpec((64, 10), lambda: (0, 0)),
                  pl.BlockSpec((1, 10), lambda: (0, 0))],
        out_specs=pl.BlockSpec((NG, 10), lambda: (0, 0)),
        out_shape=jax.ShapeDtypeStruct((NG, 10), jnp.float32),
    )(h3, bat, fc_w, fc_b)


@functools.lru_cache(maxsize=None)
def _make_sc(cout, has_cnt):
    M = cout + 16 if has_cnt else cout
    mesh = plsc.VectorSubcoreMesh(core_axis_name="c", subcore_axis_name="s")

    @functools.partial(
        pl.kernel,
        out_type=jax.ShapeDtypeStruct((2, N_PAD, M), jnp.float32),
        mesh=mesh,
        compiler_params=pltpu.CompilerParams(
            needs_layout_passes=False, use_tc_tiling_on_sc=False),
        scratch_types=[
            pltpu.VMEM((9, B), jnp.int32),
            pltpu.VMEM((9, B), jnp.int32),
            pltpu.VMEM((2, 2, B), jnp.int32),
            pltpu.VMEM((B, cout), jnp.float32),
            pltpu.VMEM((B, cout), jnp.float32),
            pltpu.VMEM((B, cout), jnp.float32),
            pltpu.VMEM((B, cout), jnp.float32),
            pltpu.VMEM((B, cout), jnp.float32),
            pltpu.VMEM((B, cout), jnp.float32),
            pltpu.VMEM((B, cout), jnp.float32),
            pltpu.VMEM((B, cout), jnp.float32),
            pltpu.VMEM((B, M), jnp.float32),
            pltpu.VMEM((B, M), jnp.float32),
            pltpu.VMEM_SHARED((N_PAD, M), jnp.float32),
            pltpu.SemaphoreType.DMA,
            pltpu.SemaphoreType.DMA,
            pltpu.SemaphoreType.DMA,
            pltpu.SemaphoreType.DMA,
            pltpu.SemaphoreType.DMA,
            pltpu.SemaphoreType.DMA,
        ],
    )
    def sc(Y, eallh, dsth, zerosh, out,
           ea0, ea1, dstv,
           r00, r01, r02, r03, r10, r11, r12, r13,
           m0, m1, agg, semg0, semg1, seme0, seme1, sems0, sems1):
        c = lax.axis_index("c")
        s = lax.axis_index("s")
        g = c * 16 + s
        gb0 = g * NB
        pltpu.sync_copy(zerosh.at[pl.ds(s * NSL, NSL)], agg.at[pl.ds(s * NSL, NSL)])
        plsc.subcore_barrier()

        ea = (ea0, ea1)
        rows = ((r00, r01, r02, r03), (r10, r11, r12, r13))
        msg = (m0, m1)
        semg = (semg0, semg1)
        seme = (seme0, seme1)
        sems = (sems0, sems1)

        def ecopy(bi, sl, rw, buf):
            pltpu.async_copy(eallh.at[gb0 + bi], ea[buf], seme[buf])
            pltpu.async_copy(dsth.at[gb0 + bi], dstv.at[sl, rw], seme[buf])

        def ewait(buf):
            pltpu.make_async_copy(eallh.at[0], ea[buf], seme[buf]).wait()
            pltpu.make_async_copy(dsth.at[0], dstv.at[0, 0], seme[buf]).wait()

        def gissue(buf):
            for k in range(4):
                pltpu.async_copy(Y.at[ea[buf].at[k]], rows[buf][k], semg[buf])

        def gwait(buf):
            for k in range(4):
                pltpu.make_async_copy(Y.at[pl.ds(0, B)], rows[buf][k],
                                      semg[buf]).wait()

        def swait(buf):
            pltpu.make_async_copy(zerosh.at[pl.ds(0, B)], msg[buf],
                                  sems[buf]).wait()

        ecopy(0, 0, 0, 0)
        ecopy(1, 0, 1, 1)
        ewait(0)
        gissue(0)

        zero16 = jnp.zeros((16,), jnp.int32)

        def pair(p, carry):
            for buf in range(2):
                bi2 = p * 2 + buf
                sl = p % 2
                gwait(buf)

                @pl.when(bi2 >= 2)
                def _():
                    swait(buf)

                @pl.when(bi2 + 2 < NB)
                def _():
                    ecopy(bi2 + 2, (p + 1) % 2, buf, buf)

                @pl.when(bi2 + 1 < NB)
                def _():
                    ewait(1 - buf)
                    gissue(1 - buf)

                r0, r1, r2, r3 = rows[buf]
                mg = msg[buf]
                eav = ea[buf]

                def qloop(q, carry2):
                    for j in range(16):
                        b = q * 16 + j
                        ev = zero16 + b
                        f32b = lambda k: plsc.bitcast(
                            plsc.load_gather(eav.at[k], [ev]), jnp.float32)
                        w0b = f32b(4)
                        w1b = f32b(5)
                        w2b = f32b(6)
                        w3b = f32b(7)
                        for t in range(cout // 16):
                            slc = pl.ds(t * 16, 16)
                            m = (w0b * r0[b, slc] + w1b * r1[b, slc]
                                 + w2b * r2[b, slc] + w3b * r3[b, slc])
                            mg[b, slc] = m
                        if has_cnt:
                            mg[b, pl.ds(cout, 16)] = f32b(8)
                    return carry2

                lax.fori_loop(0, B // 16, qloop, 0)
                pltpu.async_copy(mg, agg.at[dstv.at[sl, buf]], sems[buf],
                                 add=True)
            return carry

        lax.fori_loop(0, NB // 2, pair, 0)
        swait(0)
        swait(1)
        plsc.subcore_barrier()
        pltpu.sync_copy(agg.at[pl.ds(s * NSL, NSL)],
                        out.at[c, pl.ds(s * NSL, NSL)])

    return sc


def kernel(x, position, edge_index, pseudo, batch, W1, root1, b1,
           W2, root2, b2, W3, root3, b3, fc_w, fc_b):
    f32 = jnp.float32
    src = edge_index[0].astype(jnp.int32)
    dst = edge_index[1].astype(jnp.int32)
    pe = E_PAD - E
    p0 = jnp.pad(pseudo[:, 0].astype(f32), (0, pe)).reshape(ROWS, 128)
    p1 = jnp.pad(pseudo[:, 1].astype(f32), (0, pe)).reshape(ROWS, 128)
    src2 = jnp.pad(src, (0, pe)).reshape(ROWS, 128)
    dst2 = jnp.pad(dst, (0, pe)).reshape(ROWS, 128)

    eall = _prep(p0, p1, src2)

    xp = jnp.pad(x.astype(f32), ((0, N_PAD - N), (0, 0)))
    z48 = jnp.zeros((N_PAD, 48), f32)
    z64 = jnp.zeros((N_PAD, 64), f32)
    w1f = W1.transpose(1, 0, 2).reshape(1, KK * 32)
    w2f = W2.transpose(1, 0, 2).reshape(32, KK * 64)
    w3f = W3.transpose(1, 0, 2).reshape(64, KK * 64)
    sc1 = _make_sc(32, True)
    sc23 = _make_sc(64, False)

    Y1 = _mm(xp, w1f).reshape(N_PAD * KK, 32)
    a1 = sc1(Y1, eall, dst2, z48)
    h1, deg = _post1(a1, xp, root1, b1.reshape(1, 32))

    Y2 = _mm(h1, w2f).reshape(N_PAD * KK, 64)
    a2 = sc23(Y2, eall, dst2, z64)
    h2 = _post(a2, deg, h1, root2, b2.reshape(1, 64))

    Y3 = _mm(h2, w3f).reshape(N_PAD * KK, 64)
    a3 = sc23(Y3, eall, dst2, z64)
    h3 = _post(a3, deg, h2, root3, b3.reshape(1, 64))

    bat = jnp.pad(batch.astype(jnp.int32), (0, N_PAD - N),
                  constant_values=NG).reshape(1, N_PAD)
    return _pool(h3, bat, fc_w, fc_b.reshape(1, 10))


# speedup vs baseline: 1.1731x; 1.1731x over previous
import functools

import jax
import jax.numpy as jnp
from jax import lax
from jax.experimental import pallas as pl
from jax.experimental.pallas import tpu as pltpu
from jax.experimental.pallas import tpu_sc as plsc

K = 5
KK = K * K
N = 10000
E = 160000
NG = 64
N_PAD = 10240
E_PAD = 163840
NW = 32
EW = E_PAD // NW
B = 128
NB = EW // B
ROWS = E_PAD // B
VROWS = E // B
NSL = N_PAD // 16


def _prep(p0, p1, src, dst):
    def body(p0_r, p1_r, src_r, dst_r, e_r):
        i = pl.program_id(0)
        rows = i * 128 + lax.broadcasted_iota(jnp.int32, (128, 128), 0)
        val = (rows < VROWS).astype(jnp.float32)
        a = p0_r[...] * (K - 1.0)
        ia = jnp.clip(jnp.floor(a), 0.0, K - 2.0)
        fa = a - ia
        b = p1_r[...] * (K - 1.0)
        ib = jnp.clip(jnp.floor(b), 0.0, K - 2.0)
        fb = b - ib
        bc = lambda v: lax.bitcast_convert_type(v, jnp.int32)
        base = src_r[...] * KK + ia.astype(jnp.int32) * K + ib.astype(jnp.int32)
        e_r[:, 0, :] = base
        e_r[:, 1, :] = base + 1
        e_r[:, 2, :] = base + K
        e_r[:, 3, :] = base + K + 1
        e_r[:, 4, :] = dst_r[...]
        e_r[:, 5, :] = bc((1.0 - fa) * (1.0 - fb) * val)
        e_r[:, 6, :] = bc((1.0 - fa) * fb * val)
        e_r[:, 7, :] = bc(fa * (1.0 - fb) * val)
        e_r[:, 8, :] = bc(fa * fb * val)
        e_r[:, 9, :] = bc(val)

    spec = pl.BlockSpec((128, 128), lambda i: (i, 0))
    ospec = pl.BlockSpec((128, 10, 128), lambda i: (i, 0, 0))
    return pl.pallas_call(
        body, grid=(ROWS // 128,),
        in_specs=[spec] * 4, out_specs=ospec,
        out_shape=jax.ShapeDtypeStruct((ROWS, 10, 128), jnp.int32),
    )(p0, p1, src, dst)


def _mm(h, wflat):
    n, cin = h.shape
    cols = wflat.shape[1]
    blk = 512

    def body(h_r, w_r, o_r):
        o_r[...] = jnp.dot(h_r[...], w_r[...], preferred_element_type=jnp.float32)

    return pl.pallas_call(
        body, grid=(n // blk,),
        in_specs=[pl.BlockSpec((blk, cin), lambda i: (i, 0)),
                  pl.BlockSpec((cin, cols), lambda i: (0, 0))],
        out_specs=pl.BlockSpec((blk, cols), lambda i: (i, 0)),
        out_shape=jax.ShapeDtypeStruct((n, cols), jnp.float32),
    )(h, wflat)


def _post1(aggp, h_prev, root, bias):
    cout = root.shape[1]
    M = aggp.shape[2]
    cin = h_prev.shape[1]
    blk = 1024

    def body(a_r, h_r, r_r, b_r, o_r, d_r):
        a = a_r[0] + a_r[1]
        feat = a[:, :cout]
        cnt = a[:, cout:cout + 1]
        v = (feat / jnp.maximum(cnt, 1.0)
             + jnp.dot(h_r[...], r_r[...], preferred_element_type=jnp.float32)
             + b_r[...])
        o_r[...] = jnp.where(v > 0.0, v, jnp.exp(jnp.minimum(v, 0.0)) - 1.0)
        d_r[...] = cnt

    return pl.pallas_call(
        body, grid=(N_PAD // blk,),
        in_specs=[
            pl.BlockSpec((2, blk, M), lambda i: (0, i, 0)),
            pl.BlockSpec((blk, cin), lambda i: (i, 0)),
            pl.BlockSpec((cin, cout), lambda i: (0, 0)),
            pl.BlockSpec((1, cout), lambda i: (0, 0)),
        ],
        out_specs=[pl.BlockSpec((blk, cout), lambda i: (i, 0)),
                   pl.BlockSpec((blk, 1), lambda i: (i, 0))],
        out_shape=[jax.ShapeDtypeStruct((N_PAD, cout), jnp.float32),
                   jax.ShapeDtypeStruct((N_PAD, 1), jnp.float32)],
    )(aggp, h_prev, root, bias)


def _post(aggp, deg, h_prev, root, bias):
    cout = root.shape[1]
    M = aggp.shape[2]
    cin = h_prev.shape[1]
    blk = 1024

    def body(a_r, d_r, h_r, r_r, b_r, o_r):
        feat = a_r[0] + a_r[1]
        v = (feat / jnp.maximum(d_r[...], 1.0)
             + jnp.dot(h_r[...], r_r[...], preferred_element_type=jnp.float32)
             + b_r[...])
        o_r[...] = jnp.where(v > 0.0, v, jnp.exp(jnp.minimum(v, 0.0)) - 1.0)

    return pl.pallas_call(
        body, grid=(N_PAD // blk,),
        in_specs=[
            pl.BlockSpec((2, blk, M), lambda i: (0, i, 0)),
            pl.BlockSpec((blk, 1), lambda i: (i, 0)),
            pl.BlockSpec((blk, cin), lambda i: (i, 0)),
            pl.BlockSpec((cin, cout), lambda i: (0, 0)),
            pl.BlockSpec((1, cout), lambda i: (0, 0)),
        ],
        out_specs=pl.BlockSpec((blk, cout), lambda i: (i, 0)),
        out_shape=jax.ShapeDtypeStruct((N_PAD, cout), jnp.float32),
    )(aggp, deg, h_prev, root, bias)


def _pool(h3, bat, fc_w, fc_b):
    def body(h_r, b_r, w_r, fb_r, o_r):
        gi = lax.broadcasted_iota(jnp.int32, (NG, N_PAD), 0)
        oh = (gi == b_r[...]).astype(jnp.float32)
        seg = jnp.dot(oh, h_r[...], preferred_element_type=jnp.float32)
        cnt = jnp.sum(oh, axis=1, keepdims=True)
        g = seg / jnp.maximum(cnt, 1.0)
        logits = jnp.dot(g, w_r[...], preferred_element_type=jnp.float32) + fb_r[...]
        m = jnp.max(logits, axis=1, keepdims=True)
        lse = jnp.log(jnp.sum(jnp.exp(logits - m), axis=1, keepdims=True)) + m
        o_r[...] = logits - lse

    return pl.pallas_call(
        body,
        in_specs=[pl.BlockSpec((N_PAD, 64), lambda: (0, 0)),
                  pl.BlockSpec((1, N_PAD), lambda: (0, 0)),
                  pl.BlockSpec((64, 10), lambda: (0, 0)),
                  pl.BlockSpec((1, 10), lambda: (0, 0))],
        out_specs=pl.BlockSpec((NG, 10), lambda: (0, 0)),
        out_shape=jax.ShapeDtypeStruct((NG, 10), jnp.float32),
    )(h3, bat, fc_w, fc_b)


@functools.lru_cache(maxsize=None)
def _make_sc(cout, has_cnt):
    M = cout + 16 if has_cnt else cout
    mesh = plsc.VectorSubcoreMesh(core_axis_name="c", subcore_axis_name="s")

    @functools.partial(
        pl.kernel,
        out_type=jax.ShapeDtypeStruct((2, N_PAD, M), jnp.float32),
        mesh=mesh,
        compiler_params=pltpu.CompilerParams(
            needs_layout_passes=False, use_tc_tiling_on_sc=False),
        scratch_types=[
            pltpu.VMEM((10, B), jnp.int32),
            pltpu.VMEM((10, B), jnp.int32),
            pltpu.VMEM((B, cout), jnp.float32),
            pltpu.VMEM((B, cout), jnp.float32),
            pltpu.VMEM((B, cout), jnp.float32),
            pltpu.VMEM((B, cout), jnp.float32),
            pltpu.VMEM((B, cout), jnp.float32),
            pltpu.VMEM((B, cout), jnp.float32),
            pltpu.VMEM((B, cout), jnp.float32),
            pltpu.VMEM((B, cout), jnp.float32),
            pltpu.VMEM((B, M), jnp.float32),
            pltpu.VMEM_SHARED((N_PAD, M), jnp.float32),
            pltpu.SemaphoreType.DMA,
            pltpu.SemaphoreType.DMA,
        ],
    )
    def sc(Y, eallh, zerosh, out,
           ea0, ea1,
           r00, r01, r02, r03, r10, r11, r12, r13,
           msg, agg, sem0, sem1):
        c = lax.axis_index("c")
        s = lax.axis_index("s")
        g = c * 16 + s
        gb0 = g * NB
        pltpu.sync_copy(zerosh.at[pl.ds(s * NSL, NSL)], agg.at[pl.ds(s * NSL, NSL)])
        plsc.subcore_barrier()

        bufs = ((ea0, (r00, r01, r02, r03), sem0),
                (ea1, (r10, r11, r12, r13), sem1))

        def fetch(gb, eav, rows, sem):
            pltpu.sync_copy(eallh.at[gb], eav)
            for k in range(4):
                pltpu.async_copy(Y.at[eav.at[k]], rows[k], sem)

        def gwait(rows, sem):
            for k in range(4):
                pltpu.make_async_copy(Y.at[pl.ds(0, B)], rows[k], sem).wait()

        fetch(gb0, *bufs[0])

        zero16 = jnp.zeros((16,), jnp.int32)

        def pair(pp, carry):
            for buf in range(2):
                bi = pp * 2 + buf
                eav, rows, sem = bufs[buf]
                neav, nrows, nsem = bufs[1 - buf]

                @pl.when(bi + 1 < NB)
                def _():
                    fetch(gb0 + bi + 1, neav, nrows, nsem)

                gwait(rows, sem)
                r0, r1, r2, r3 = rows

                def qloop(q, carry2):
                    for j in range(16):
                        b = q * 16 + j
                        ev = zero16 + b
                        f32b = lambda k: plsc.bitcast(
                            plsc.load_gather(eav.at[k], [ev]), jnp.float32)
                        w0b = f32b(5)
                        w1b = f32b(6)
                        w2b = f32b(7)
                        w3b = f32b(8)
                        for t in range(cout // 16):
                            slc = pl.ds(t * 16, 16)
                            m = (w0b * r0[b, slc] + w1b * r1[b, slc]
                                 + w2b * r2[b, slc] + w3b * r3[b, slc])
                            msg[b, slc] = m
                        if has_cnt:
                            msg[b, pl.ds(cout, 16)] = f32b(9)
                    return carry2

                lax.fori_loop(0, B // 16, qloop, 0)
                pltpu.sync_copy(msg, agg.at[eav.at[4]], add=True)
            return carry

        lax.fori_loop(0, NB // 2, pair, 0)
        plsc.subcore_barrier()
        pltpu.sync_copy(agg.at[pl.ds(s * NSL, NSL)],
                        out.at[c, pl.ds(s * NSL, NSL)])

    return sc


def kernel(x, position, edge_index, pseudo, batch, W1, root1, b1,
           W2, root2, b2, W3, root3, b3, fc_w, fc_b):
    f32 = jnp.float32
    src = edge_index[0].astype(jnp.int32)
    dst = edge_index[1].astype(jnp.int32)
    pe = E_PAD - E
    p0 = jnp.pad(pseudo[:, 0].astype(f32), (0, pe)).reshape(ROWS, 128)
    p1 = jnp.pad(pseudo[:, 1].astype(f32), (0, pe)).reshape(ROWS, 128)
    src2 = jnp.pad(src, (0, pe)).reshape(ROWS, 128)
    dst2 = jnp.pad(dst, (0, pe)).reshape(ROWS, 128)

    eall = _prep(p0, p1, src2, dst2)

    xp = jnp.pad(x.astype(f32), ((0, N_PAD - N), (0, 0)))
    z48 = jnp.zeros((N_PAD, 48), f32)
    z64 = jnp.zeros((N_PAD, 64), f32)
    w1f = W1.transpose(1, 0, 2).reshape(1, KK * 32)
    w2f = W2.transpose(1, 0, 2).reshape(32, KK * 64)
    w3f = W3.transpose(1, 0, 2).reshape(64, KK * 64)
    sc1 = _make_sc(32, True)
    sc23 = _make_sc(64, False)

    Y1 = _mm(xp, w1f).reshape(N_PAD * KK, 32)
    a1 = sc1(Y1, eall, z48)
    h1, deg = _post1(a1, xp, root1, b1.reshape(1, 32))

    Y2 = _mm(h1, w2f).reshape(N_PAD * KK, 64)
    a2 = sc23(Y2, eall, z64)
    h2 = _post(a2, deg, h1, root2, b2.reshape(1, 64))

    Y3 = _mm(h2, w3f).reshape(N_PAD * KK, 64)
    a3 = sc23(Y3, eall, z64)
    h3 = _post(a3, deg, h2, root3, b3.reshape(1, 64))

    bat = jnp.pad(batch.astype(jnp.int32), (0, N_PAD - N),
                  constant_values=NG).reshape(1, N_PAD)
    return _pool(h3, bat, fc_w, fc_b.reshape(1, 10))


# speedup vs baseline: 1.5593x; 1.3292x over previous
import functools

import jax
import jax.numpy as jnp
from jax import lax
from jax.experimental import pallas as pl
from jax.experimental.pallas import tpu as pltpu
from jax.experimental.pallas import tpu_sc as plsc

K = 5
KK = K * K
N = 10000
E = 160000
NG = 64
N_PAD = 10240
E_PAD = 163840
NW = 32
EW = E_PAD // NW
B = 128
NB = EW // B
ROWS = E_PAD // B
VROWS = E // B
NSL = N_PAD // 16


def _prep(p0, p1, src, dst):
    def body(p0_r, p1_r, src_r, dst_r, e_r):
        i = pl.program_id(0)
        rows = i * 128 + lax.broadcasted_iota(jnp.int32, (128, 128), 0)
        val = (rows < VROWS).astype(jnp.float32)
        a = p0_r[...] * (K - 1.0)
        ia = jnp.clip(jnp.floor(a), 0.0, K - 2.0)
        fa = a - ia
        b = p1_r[...] * (K - 1.0)
        ib = jnp.clip(jnp.floor(b), 0.0, K - 2.0)
        fb = b - ib
        bc = lambda v: lax.bitcast_convert_type(v, jnp.int32)
        base = src_r[...] * KK + ia.astype(jnp.int32) * K + ib.astype(jnp.int32)
        e_r[:, 0, :] = base
        e_r[:, 1, :] = base + 1
        e_r[:, 2, :] = base + K
        e_r[:, 3, :] = base + K + 1
        e_r[:, 4, :] = dst_r[...]
        e_r[:, 5, :] = bc((1.0 - fa) * (1.0 - fb) * val)
        e_r[:, 6, :] = bc((1.0 - fa) * fb * val)
        e_r[:, 7, :] = bc(fa * (1.0 - fb) * val)
        e_r[:, 8, :] = bc(fa * fb * val)
        e_r[:, 9, :] = bc(val)

    spec = pl.BlockSpec((128, 128), lambda i: (i, 0))
    ospec = pl.BlockSpec((128, 10, 128), lambda i: (i, 0, 0))
    return pl.pallas_call(
        body, grid=(ROWS // 128,),
        in_specs=[spec] * 4, out_specs=ospec,
        out_shape=jax.ShapeDtypeStruct((ROWS, 10, 128), jnp.int32),
    )(p0, p1, src, dst)


def _mm(h, wflat):
    n, cin = h.shape
    cols = wflat.shape[1]
    blk = 512

    def body(h_r, w_r, o_r):
        o_r[...] = jnp.dot(h_r[...], w_r[...], preferred_element_type=jnp.float32)

    return pl.pallas_call(
        body, grid=(n // blk,),
        in_specs=[pl.BlockSpec((blk, cin), lambda i: (i, 0)),
                  pl.BlockSpec((cin, cols), lambda i: (0, 0))],
        out_specs=pl.BlockSpec((blk, cols), lambda i: (i, 0)),
        out_shape=jax.ShapeDtypeStruct((n, cols), jnp.float32),
    )(h, wflat)


def _post1(aggp, h_prev, root, bias):
    cout = root.shape[1]
    M = aggp.shape[2]
    cin = h_prev.shape[1]
    blk = 1024

    def body(a_r, h_r, r_r, b_r, o_r, d_r):
        a = a_r[0] + a_r[1]
        feat = a[:, :cout]
        cnt = a[:, cout:cout + 1]
        v = (feat / jnp.maximum(cnt, 1.0)
             + jnp.dot(h_r[...], r_r[...], preferred_element_type=jnp.float32)
             + b_r[...])
        o_r[...] = jnp.where(v > 0.0, v, jnp.exp(jnp.minimum(v, 0.0)) - 1.0)
        d_r[...] = cnt

    return pl.pallas_call(
        body, grid=(N_PAD // blk,),
        in_specs=[
            pl.BlockSpec((2, blk, M), lambda i: (0, i, 0)),
            pl.BlockSpec((blk, cin), lambda i: (i, 0)),
            pl.BlockSpec((cin, cout), lambda i: (0, 0)),
            pl.BlockSpec((1, cout), lambda i: (0, 0)),
        ],
        out_specs=[pl.BlockSpec((blk, cout), lambda i: (i, 0)),
                   pl.BlockSpec((blk, 1), lambda i: (i, 0))],
        out_shape=[jax.ShapeDtypeStruct((N_PAD, cout), jnp.float32),
                   jax.ShapeDtypeStruct((N_PAD, 1), jnp.float32)],
    )(aggp, h_prev, root, bias)


def _post(aggp, deg, h_prev, root, bias):
    cout = root.shape[1]
    M = aggp.shape[2]
    cin = h_prev.shape[1]
    blk = 1024

    def body(a_r, d_r, h_r, r_r, b_r, o_r):
        feat = a_r[0] + a_r[1]
        v = (feat / jnp.maximum(d_r[...], 1.0)
             + jnp.dot(h_r[...], r_r[...], preferred_element_type=jnp.float32)
             + b_r[...])
        o_r[...] = jnp.where(v > 0.0, v, jnp.exp(jnp.minimum(v, 0.0)) - 1.0)

    return pl.pallas_call(
        body, grid=(N_PAD // blk,),
        in_specs=[
            pl.BlockSpec((2, blk, M), lambda i: (0, i, 0)),
            pl.BlockSpec((blk, 1), lambda i: (i, 0)),
            pl.BlockSpec((blk, cin), lambda i: (i, 0)),
            pl.BlockSpec((cin, cout), lambda i: (0, 0)),
            pl.BlockSpec((1, cout), lambda i: (0, 0)),
        ],
        out_specs=pl.BlockSpec((blk, cout), lambda i: (i, 0)),
        out_shape=jax.ShapeDtypeStruct((N_PAD, cout), jnp.float32),
    )(aggp, deg, h_prev, root, bias)


def _post1_mm(aggp, h_prev, root, bias, wflat):
    cout = root.shape[1]
    M = aggp.shape[2]
    cin = h_prev.shape[1]
    cols = wflat.shape[1]
    blk = 1024

    def body(a_r, h_r, r_r, b_r, w_r, o_r, d_r, y_r):
        a = a_r[0] + a_r[1]
        feat = a[:, :cout]
        cnt = a[:, cout:cout + 1]
        v = (feat / jnp.maximum(cnt, 1.0)
             + jnp.dot(h_r[...], r_r[...], preferred_element_type=jnp.float32)
             + b_r[...])
        h = jnp.where(v > 0.0, v, jnp.exp(jnp.minimum(v, 0.0)) - 1.0)
        o_r[...] = h
        d_r[...] = cnt
        y_r[...] = jnp.dot(h, w_r[...], preferred_element_type=jnp.float32)

    return pl.pallas_call(
        body, grid=(N_PAD // blk,),
        in_specs=[
            pl.BlockSpec((2, blk, M), lambda i: (0, i, 0)),
            pl.BlockSpec((blk, cin), lambda i: (i, 0)),
            pl.BlockSpec((cin, cout), lambda i: (0, 0)),
            pl.BlockSpec((1, cout), lambda i: (0, 0)),
            pl.BlockSpec((cout, cols), lambda i: (0, 0)),
        ],
        out_specs=[pl.BlockSpec((blk, cout), lambda i: (i, 0)),
                   pl.BlockSpec((blk, 1), lambda i: (i, 0)),
                   pl.BlockSpec((blk, cols), lambda i: (i, 0))],
        out_shape=[jax.ShapeDtypeStruct((N_PAD, cout), jnp.float32),
                   jax.ShapeDtypeStruct((N_PAD, 1), jnp.float32),
                   jax.ShapeDtypeStruct((N_PAD, cols), jnp.float32)],
    )(aggp, h_prev, root, bias, wflat)


def _post_mm(aggp, deg, h_prev, root, bias, wflat):
    cout = root.shape[1]
    M = aggp.shape[2]
    cin = h_prev.shape[1]
    cols = wflat.shape[1]
    blk = 1024

    def body(a_r, d_r, h_r, r_r, b_r, w_r, o_r, y_r):
        feat = a_r[0] + a_r[1]
        v = (feat / jnp.maximum(d_r[...], 1.0)
             + jnp.dot(h_r[...], r_r[...], preferred_element_type=jnp.float32)
             + b_r[...])
        h = jnp.where(v > 0.0, v, jnp.exp(jnp.minimum(v, 0.0)) - 1.0)
        o_r[...] = h
        y_r[...] = jnp.dot(h, w_r[...], preferred_element_type=jnp.float32)

    return pl.pallas_call(
        body, grid=(N_PAD // blk,),
        in_specs=[
            pl.BlockSpec((2, blk, M), lambda i: (0, i, 0)),
            pl.BlockSpec((blk, 1), lambda i: (i, 0)),
            pl.BlockSpec((blk, cin), lambda i: (i, 0)),
            pl.BlockSpec((cin, cout), lambda i: (0, 0)),
            pl.BlockSpec((1, cout), lambda i: (0, 0)),
            pl.BlockSpec((cout, cols), lambda i: (0, 0)),
        ],
        out_specs=[pl.BlockSpec((blk, cout), lambda i: (i, 0)),
                   pl.BlockSpec((blk, cols), lambda i: (i, 0))],
        out_shape=[jax.ShapeDtypeStruct((N_PAD, cout), jnp.float32),
                   jax.ShapeDtypeStruct((N_PAD, cols), jnp.float32)],
    )(aggp, deg, h_prev, root, bias, wflat)


def _pool(h3, bat, fc_w, fc_b):
    def body(h_r, b_r, w_r, fb_r, o_r):
        gi = lax.broadcasted_iota(jnp.int32, (NG, N_PAD), 0)
        oh = (gi == b_r[...]).astype(jnp.float32)
        seg = jnp.dot(oh, h_r[...], preferred_element_type=jnp.float32)
        cnt = jnp.sum(oh, axis=1, keepdims=True)
        g = seg / jnp.maximum(cnt, 1.0)
        logits = jnp.dot(g, w_r[...], preferred_element_type=jnp.float32) + fb_r[...]
        m = jnp.max(logits, axis=1, keepdims=True)
        lse = jnp.log(jnp.sum(jnp.exp(logits - m), axis=1, keepdims=True)) + m
        o_r[...] = logits - lse

    return pl.pallas_call(
        body,
        in_specs=[pl.BlockSpec((N_PAD, 64), lambda: (0, 0)),
                  pl.BlockSpec((1, N_PAD), lambda: (0, 0)),
                  pl.BlockSpec((64, 10), lambda: (0, 0)),
                  pl.BlockSpec((1, 10), lambda: (0, 0))],
        out_specs=pl.BlockSpec((NG, 10), lambda: (0, 0)),
        out_shape=jax.ShapeDtypeStruct((NG, 10), jnp.float32),
    )(h3, bat, fc_w, fc_b)


@functools.lru_cache(maxsize=None)
def _make_sc(cout, has_cnt):
    M = cout + 16 if has_cnt else cout
    mesh = plsc.VectorSubcoreMesh(core_axis_name="c", subcore_axis_name="s")

    @functools.partial(
        pl.kernel,
        out_type=jax.ShapeDtypeStruct((2, N_PAD, M), jnp.float32),
        mesh=mesh,
        compiler_params=pltpu.CompilerParams(
            needs_layout_passes=False, use_tc_tiling_on_sc=False),
        scratch_types=[
            pltpu.VMEM((10, B), jnp.int32),
            pltpu.VMEM((10, B), jnp.int32),
            pltpu.VMEM((B, cout), jnp.float32),
            pltpu.VMEM((B, cout), jnp.float32),
            pltpu.VMEM((B, cout), jnp.float32),
            pltpu.VMEM((B, cout), jnp.float32),
            pltpu.VMEM((B, cout), jnp.float32),
            pltpu.VMEM((B, cout), jnp.float32),
            pltpu.VMEM((B, cout), jnp.float32),
            pltpu.VMEM((B, cout), jnp.float32),
            pltpu.VMEM((B, M), jnp.float32),
            pltpu.VMEM_SHARED((N_PAD, M), jnp.float32),
            pltpu.SemaphoreType.DMA,
            pltpu.SemaphoreType.DMA,
        ],
    )
    def sc(Y, eallh, zerosh, out,
           ea0, ea1,
           r00, r01, r02, r03, r10, r11, r12, r13,
           msg, agg, sem0, sem1):
        c = lax.axis_index("c")
        s = lax.axis_index("s")
        g = c * 16 + s
        gb0 = g * NB
        pltpu.sync_copy(zerosh.at[pl.ds(s * NSL, NSL)], agg.at[pl.ds(s * NSL, NSL)])
        plsc.subcore_barrier()

        bufs = ((ea0, (r00, r01, r02, r03), sem0),
                (ea1, (r10, r11, r12, r13), sem1))

        def fetch(gb, eav, rows, sem):
            pltpu.sync_copy(eallh.at[gb], eav)
            for k in range(4):
                pltpu.async_copy(Y.at[eav.at[k]], rows[k], sem)

        def gwait(rows, sem):
            for k in range(4):
                pltpu.make_async_copy(Y.at[pl.ds(0, B)], rows[k], sem).wait()

        fetch(gb0, *bufs[0])

        zero16 = jnp.zeros((16,), jnp.int32)

        def pair(pp, carry):
            for buf in range(2):
                bi = pp * 2 + buf
                eav, rows, sem = bufs[buf]
                neav, nrows, nsem = bufs[1 - buf]

                @pl.when(bi + 1 < NB)
                def _():
                    fetch(gb0 + bi + 1, neav, nrows, nsem)

                gwait(rows, sem)
                r0, r1, r2, r3 = rows

                def qloop(q, carry2):
                    for j in range(16):
                        b = q * 16 + j
                        ev = zero16 + b
                        f32b = lambda k: plsc.bitcast(
                            plsc.load_gather(eav.at[k], [ev]), jnp.float32)
                        w0b = f32b(5)
                        w1b = f32b(6)
                        w2b = f32b(7)
                        w3b = f32b(8)
                        for t in range(cout // 16):
                            slc = pl.ds(t * 16, 16)
                            m = (w0b * r0[b, slc] + w1b * r1[b, slc]
                                 + w2b * r2[b, slc] + w3b * r3[b, slc])
                            msg[b, slc] = m
                        if has_cnt:
                            msg[b, pl.ds(cout, 16)] = f32b(9)
                    return carry2

                lax.fori_loop(0, B // 16, qloop, 0)
                pltpu.sync_copy(msg, agg.at[eav.at[4]], add=True)
            return carry

        lax.fori_loop(0, NB // 2, pair, 0)
        plsc.subcore_barrier()
        pltpu.sync_copy(agg.at[pl.ds(s * NSL, NSL)],
                        out.at[c, pl.ds(s * NSL, NSL)])

    return sc


def kernel(x, position, edge_index, pseudo, batch, W1, root1, b1,
           W2, root2, b2, W3, root3, b3, fc_w, fc_b):
    f32 = jnp.float32
    src = edge_index[0].astype(jnp.int32)
    dst = edge_index[1].astype(jnp.int32)
    pe = E_PAD - E
    p0 = jnp.pad(pseudo[:, 0].astype(f32), (0, pe)).reshape(ROWS, 128)
    p1 = jnp.pad(pseudo[:, 1].astype(f32), (0, pe)).reshape(ROWS, 128)
    tail = jnp.arange(pe, dtype=jnp.int32)
    src2 = jnp.concatenate([src, (tail * 79) % N]).reshape(ROWS, 128)
    dst2 = jnp.concatenate([dst, N + (tail % (N_PAD - N))]).reshape(ROWS, 128)

    eall = _prep(p0, p1, src2, dst2)

    xp = jnp.pad(x.astype(f32), ((0, N_PAD - N), (0, 0)))
    z48 = jnp.zeros((N_PAD, 48), f32)
    z64 = jnp.zeros((N_PAD, 64), f32)
    w1f = W1.transpose(1, 0, 2).reshape(1, KK * 32)
    w2f = W2.transpose(1, 0, 2).reshape(32, KK * 64)
    w3f = W3.transpose(1, 0, 2).reshape(64, KK * 64)
    sc1 = _make_sc(32, True)
    sc23 = _make_sc(64, False)

    Y1 = _mm(xp, w1f).reshape(N_PAD * KK, 32)
    a1 = sc1(Y1, eall, z48)
    h1, deg, Y2f = _post1_mm(a1, xp, root1, b1.reshape(1, 32), w2f)

    a2 = sc23(Y2f.reshape(N_PAD * KK, 64), eall, z64)
    h2, Y3f = _post_mm(a2, deg, h1, root2, b2.reshape(1, 64), w3f)

    a3 = sc23(Y3f.reshape(N_PAD * KK, 64), eall, z64)
    h3 = _post(a3, deg, h2, root3, b3.reshape(1, 64))

    bat = jnp.pad(batch.astype(jnp.int32), (0, N_PAD - N),
                  constant_values=NG).reshape(1, N_PAD)
    return _pool(h3, bat, fc_w, fc_b.reshape(1, 10))
